# baseline (speedup 1.0000x reference)
import functools

import jax
import jax.numpy as jnp
from jax import lax
from jax.experimental import pallas as pl
from jax.experimental.pallas import tpu as pltpu
from jax.experimental.pallas import tpu_sc as plsc

H = 128
NB = 6
NM = 1000
L = 50
NC, NS = 2, 16
NW = NC * NS

AP = 51200
BP = 200704

A_PW, A_SB, A_NSB = 1600, 16, 100
NB8 = 8
B_PW, B_SB, B_NSB = 6272, 128, 49

F32 = jnp.float32


DEPTH_RING = 3


def _sc_atom_body(add_matom, mb_hbm, ma_hbm, a2b_hbm, out_hbm,
                  idx_v, r0_v, r1_v, r2_v, ma0_v, ma1_v, ma2_v,
                  o0_v, o1_v, o2_v, g0, g1, g2, s0, s1, s2):
    wid = lax.axis_index("s") * NC + lax.axis_index("c")
    ibase = wid * A_PW * NB8
    pltpu.sync_copy(a2b_hbm.at[pl.ds(ibase, A_PW * NB8)], idx_v)
    rows = (r0_v, r1_v, r2_v)
    outs = (o0_v, o1_v, o2_v)
    mas = (ma0_v, ma1_v, ma2_v)
    gsem = (g0, g1, g2)
    ssem = (s0, s1, s2)
    nidx = A_SB * NB8

    def fire(i, b):
        pltpu.async_copy(mb_hbm.at[idx_v.at[pl.ds(i * nidx, nidx)]],
                         rows[b], gsem[b])
        if add_matom:
            pltpu.async_copy(ma_hbm.at[pl.ds(wid * A_PW + i * A_SB, A_SB)],
                             mas[b], gsem[b])

    def compute(i, b):
        abase = wid * A_PW + i * A_SB
        @pl.when(i >= DEPTH_RING)
        def _():
            pltpu.make_async_copy(outs[b], out_hbm.at[pl.ds(abase, A_SB)],
                                  ssem[b]).wait()
        pltpu.make_async_copy(mb_hbm.at[pl.ds(0, nidx)], rows[b],
                              gsem[b]).wait()
        if add_matom:
            pltpu.make_async_copy(ma_hbm.at[pl.ds(0, A_SB)], mas[b],
                                  gsem[b]).wait()
        o_v = outs[b]
        ma_v = mas[b]
        r_v = rows[b]

        def per_atom(a, carry):
            for ch in range(H // 16):
                sl = pl.ds(ch * 16, 16)
                v = r_v[a * NB8, sl]
                ssum = v
                smax = v
                for j in range(1, NB):
                    vj = r_v[a * NB8 + j, sl]
                    ssum = ssum + vj
                    smax = jnp.maximum(smax, vj)
                res = ssum * smax
                if add_matom:
                    res = res + ma_v[a, sl]
                o_v[a, sl] = res
            return carry

        lax.fori_loop(0, A_SB, per_atom, 0)
        pltpu.async_copy(o_v, out_hbm.at[pl.ds(abase, A_SB)], ssem[b])

    for p in range(DEPTH_RING - 1):
        fire(p, p)

    def step3(i3, carry):
        for k in range(DEPTH_RING):
            i = i3 * DEPTH_RING + k

            @pl.when(i + DEPTH_RING - 1 < A_NSB)
            def _():
                fire(i + DEPTH_RING - 1, (k + DEPTH_RING - 1) % DEPTH_RING)

            compute(i, k)
        return carry

    lax.fori_loop(0, A_NSB // DEPTH_RING, step3, 0)
    tail = (A_NSB // DEPTH_RING) * DEPTH_RING
    for i in range(tail, A_NSB):
        compute(i, i % DEPTH_RING)
    for i in range(A_NSB - DEPTH_RING, A_NSB):
        b = i % DEPTH_RING
        abase = wid * A_PW + i * A_SB
        pltpu.make_async_copy(outs[b], out_hbm.at[pl.ds(abase, A_SB)],
                              ssem[b]).wait()


def _sc_atom(mb, ma, a2b_flat, add_matom):
    return pl.kernel(
        functools.partial(_sc_atom_body, add_matom),
        out_type=jax.ShapeDtypeStruct((AP, H), F32),
        mesh=plsc.VectorSubcoreMesh(core_axis_name="c", subcore_axis_name="s"),
        scratch_types=[
            pltpu.VMEM((A_PW * NB8,), jnp.int32),
            pltpu.VMEM((A_SB * NB8, H), F32),
            pltpu.VMEM((A_SB * NB8, H), F32),
            pltpu.VMEM((A_SB * NB8, H), F32),
            pltpu.VMEM((A_SB, H), F32),
            pltpu.VMEM((A_SB, H), F32),
            pltpu.VMEM((A_SB, H), F32),
            pltpu.VMEM((A_SB, H), F32),
            pltpu.VMEM((A_SB, H), F32),
            pltpu.VMEM((A_SB, H), F32),
            pltpu.SemaphoreType.DMA,
            pltpu.SemaphoreType.DMA,
            pltpu.SemaphoreType.DMA,
            pltpu.SemaphoreType.DMA,
            pltpu.SemaphoreType.DMA,
            pltpu.SemaphoreType.DMA,
        ],
    )(mb, ma, a2b_flat)


def _sc_bond_body(ma_hbm, mb_hbm, b2a_hbm, b2r_hbm, out_hbm,
                  idxa_v, idxr_v, a0_v, a1_v, r0_v, r1_v, o0_v, o1_v,
                  g0, g1, s0, s1):
    wid = lax.axis_index("s") * NC + lax.axis_index("c")
    ibase = wid * B_PW
    pltpu.sync_copy(b2a_hbm.at[pl.ds(ibase, B_PW)], idxa_v)
    pltpu.sync_copy(b2r_hbm.at[pl.ds(ibase, B_PW)], idxr_v)
    ags = (a0_v, a1_v)
    rvs = (r0_v, r1_v)
    outs = (o0_v, o1_v)
    gsem = (g0, g1)
    ssem = (s0, s1)

    def fire(i, b):
        off = i * B_SB
        pltpu.async_copy(ma_hbm.at[idxa_v.at[pl.ds(off, B_SB)]], ags[b], gsem[b])
        pltpu.async_copy(mb_hbm.at[idxr_v.at[pl.ds(off, B_SB)]], rvs[b], gsem[b])

    def compute(i, b):
        bbase = wid * B_PW + i * B_SB

        @pl.when(i >= 2)
        def _():
            pltpu.make_async_copy(outs[b], out_hbm.at[pl.ds(bbase, B_SB)],
                                  ssem[b]).wait()

        desc = pltpu.make_async_copy(ma_hbm.at[pl.ds(0, B_SB)], ags[b], gsem[b])
        desc.wait()
        desc.wait()
        o_v = outs[b]

        def inner(r, c):
            for ch in range(H // 16):
                sl = pl.ds(ch * 16, 16)
                o_v[r, sl] = ags[b][r, sl] - rvs[b][r, sl]
            return c

        lax.fori_loop(0, B_SB, inner, 0)
        pltpu.async_copy(o_v, out_hbm.at[pl.ds(bbase, B_SB)], ssem[b])

    fire(0, 0)

    def step2(i2, carry):
        for b in range(2):
            i = i2 * 2 + b

            @pl.when(i + 1 < B_NSB)
            def _():
                fire(i + 1, (b + 1) % 2)

            @pl.when(i < B_NSB)
            def _():
                compute(i, b)
        return carry

    lax.fori_loop(0, (B_NSB + 1) // 2, step2, 0)
    for b in range(2):
        bbase = wid * B_PW + (B_NSB - 2 + b) * B_SB
        pltpu.make_async_copy(outs[b], out_hbm.at[pl.ds(bbase, B_SB)],
                              ssem[b]).wait()


def _sc_bond(ma, mb, b2a_flat, b2r_flat):
    return pl.kernel(
        _sc_bond_body,
        out_type=jax.ShapeDtypeStruct((BP, H), F32),
        mesh=plsc.VectorSubcoreMesh(core_axis_name="c", subcore_axis_name="s"),
        scratch_types=[
            pltpu.VMEM((B_PW,), jnp.int32),
            pltpu.VMEM((B_PW,), jnp.int32),
            pltpu.VMEM((B_SB, H), F32),
            pltpu.VMEM((B_SB, H), F32),
            pltpu.VMEM((B_SB, H), F32),
            pltpu.VMEM((B_SB, H), F32),
            pltpu.VMEM((B_SB, H), F32),
            pltpu.VMEM((B_SB, H), F32),
            pltpu.SemaphoreType.DMA,
            pltpu.SemaphoreType.DMA,
            pltpu.SemaphoreType.DMA,
            pltpu.SemaphoreType.DMA,
        ],
    )(ma, mb, b2a_flat, b2r_flat)


def _relu_mm_body(x_ref, w_ref, o_ref):
    o_ref[...] = jax.nn.relu(
        jnp.dot(x_ref[...], w_ref[...], preferred_element_type=F32))


def _relu_mm(x, w, blk):
    n, k = x.shape
    m = w.shape[1]
    return pl.pallas_call(
        _relu_mm_body,
        grid=(n // blk,),
        in_specs=[pl.BlockSpec((blk, k), lambda i: (i, 0)),
                  pl.BlockSpec((k, m), lambda i: (0, 0))],
        out_specs=pl.BlockSpec((blk, m), lambda i: (i, 0)),
        out_shape=jax.ShapeDtypeStruct((n, m), F32),
    )(x, w)


def _relu_mm_t_body(xt_ref, w_ref, o_ref):
    o_ref[...] = jax.nn.relu(lax.dot_general(
        xt_ref[...], w_ref[...], (((0,), (0,)), ((), ())),
        preferred_element_type=F32))


def _relu_mm_t(xt, w, blk):
    k, n = xt.shape
    m = w.shape[1]
    return pl.pallas_call(
        _relu_mm_t_body,
        grid=(n // blk,),
        in_specs=[pl.BlockSpec((k, blk), lambda i: (0, i)),
                  pl.BlockSpec((k, m), lambda i: (0, 0))],
        out_specs=pl.BlockSpec((blk, m), lambda i: (i, 0)),
        out_shape=jax.ShapeDtypeStruct((n, m), F32),
    )(xt, w)


def _depth_mm_body(x_ref, w_ref, b_ref, o_ref):
    o_ref[...] = jax.nn.relu(
        b_ref[...] + jnp.dot(x_ref[...], w_ref[...], preferred_element_type=F32))


def _depth_mm(pre, w, ib, blk):
    n, k = pre.shape
    m = w.shape[1]
    return pl.pallas_call(
        _depth_mm_body,
        grid=(n // blk,),
        in_specs=[pl.BlockSpec((blk, k), lambda i: (i, 0)),
                  pl.BlockSpec((k, m), lambda i: (0, 0)),
                  pl.BlockSpec((blk, m), lambda i: (i, 0))],
        out_specs=pl.BlockSpec((blk, m), lambda i: (i, 0)),
        out_shape=jax.ShapeDtypeStruct((n, m), F32),
    )(pre, w, ib)


def _readout_body(a_ref, m_ref, i_ref, w0, w1, w2, o_ref):
    o_ref[...] = (jnp.dot(a_ref[...], w0[...], preferred_element_type=F32)
                  + jnp.dot(m_ref[...], w1[...], preferred_element_type=F32)
                  + jnp.dot(i_ref[...], w2[...], preferred_element_type=F32))


def _readout(agg, ma, ia, w_lr, blk=2048):
    n = agg.shape[0]
    w0, w1, w2 = w_lr[:H], w_lr[H:2 * H], w_lr[2 * H:]
    return pl.pallas_call(
        _readout_body,
        grid=(n // blk,),
        in_specs=[pl.BlockSpec((blk, H), lambda i: (i, 0)),
                  pl.BlockSpec((blk, H), lambda i: (i, 0)),
                  pl.BlockSpec((blk, H), lambda i: (i, 0)),
                  pl.BlockSpec((H, H), lambda i: (0, 0)),
                  pl.BlockSpec((H, H), lambda i: (0, 0)),
                  pl.BlockSpec((H, H), lambda i: (0, 0))],
        out_specs=pl.BlockSpec((blk, H), lambda i: (i, 0)),
        out_shape=jax.ShapeDtypeStruct((n, H), F32),
    )(agg, ma, ia, w0, w1, w2)


def _h0_body(hid_ref, o_ref):
    t = pl.program_id(0)

    @pl.when(t == 0)
    def _():
        o_ref[...] = hid_ref[0]

    @pl.when(t > 0)
    def _():
        o_ref[...] = jnp.maximum(o_ref[...], hid_ref[0])


def _h0(hid_tm):
    return pl.pallas_call(
        _h0_body,
        grid=(L,),
        in_specs=[pl.BlockSpec((1, NM, H), lambda t: (t, 0, 0))],
        out_specs=pl.BlockSpec((NM, H), lambda t: (0, 0)),
        out_shape=jax.ShapeDtypeStruct((NM, H), F32),
    )(hid_tm)


def _gru_body(hf_ref, hb_ref, h0_ref, wih_f, whh_f, wih_b, whh_b,
              bih_f, bhh_f, bih_b, bhh_b, gb_ref,
              of_ref, ob_ref, hf, hb):
    t = pl.program_id(0)

    @pl.when(t == 0)
    def _():
        hf[...] = h0_ref[...]
        hb[...] = h0_ref[...]

    gb = gb_ref[...]

    def cell(x, h, wih, whh, bih, bhh):
        gi = jnp.dot(x, wih[...], preferred_element_type=F32) + bih[...]
        gh = jnp.dot(h, whh[...], preferred_element_type=F32) + bhh[...]
        r = jax.nn.sigmoid(gi[:, :H] + gh[:, :H])
        z = jax.nn.sigmoid(gi[:, H:2 * H] + gh[:, H:2 * H])
        n = jnp.tanh(gi[:, 2 * H:] + r * gh[:, 2 * H:])
        return (1.0 - z) * n + z * h

    xf = jax.nn.relu(hf_ref[0] + gb)
    hfn = cell(xf, hf[...], wih_f, whh_f, bih_f, bhh_f)
    hf[...] = hfn
    of_ref[0] = hfn

    xb = jax.nn.relu(hb_ref[0] + gb)
    hbn = cell(xb, hb[...], wih_b, whh_b, bih_b, bhh_b)
    hb[...] = hbn
    ob_ref[0] = hbn


def _gru(hid_tm, h0, wih_f, whh_f, bih_f, bhh_f, wih_b, whh_b, bih_b, bhh_b,
         gru_bias):
    wspec = pl.BlockSpec((H, 3 * H), lambda t: (0, 0))
    bspec = pl.BlockSpec((1, 3 * H), lambda t: (0, 0))
    return pl.pallas_call(
        _gru_body,
        grid=(L,),
        in_specs=[pl.BlockSpec((1, NM, H), lambda t: (t, 0, 0)),
                  pl.BlockSpec((1, NM, H), lambda t: (L - 1 - t, 0, 0)),
                  pl.BlockSpec((NM, H), lambda t: (0, 0)),
                  wspec, wspec, wspec, wspec,
                  bspec, bspec, bspec, bspec,
                  pl.BlockSpec((1, H), lambda t: (0, 0))],
        out_specs=[pl.BlockSpec((1, NM, H), lambda t: (t, 0, 0)),
                   pl.BlockSpec((1, NM, H), lambda t: (t, 0, 0))],
        out_shape=[jax.ShapeDtypeStruct((L, NM, H), F32),
                   jax.ShapeDtypeStruct((L, NM, H), F32)],
        scratch_shapes=[pltpu.VMEM((NM, H), F32),
                        pltpu.VMEM((NM, H), F32)],
    )(hid_tm, hid_tm, h0,
      wih_f, whh_f, wih_b, whh_b,
      bih_f.reshape(1, 3 * H), bhh_f.reshape(1, 3 * H),
      bih_b.reshape(1, 3 * H), bhh_b.reshape(1, 3 * H),
      gru_bias.reshape(1, H))


def _mol_body(of_ref, ob_ref, w0, w1, bo_ref, o_ref):
    t = pl.program_id(0)
    contrib = jax.nn.relu(
        jnp.dot(of_ref[0], w0[...], preferred_element_type=F32)
        + jnp.dot(ob_ref[0], w1[...], preferred_element_type=F32)
        + bo_ref[...]) * (1.0 / L)

    @pl.when(t == 0)
    def _():
        o_ref[...] = contrib

    @pl.when(t > 0)
    def _():
        o_ref[...] = o_ref[...] + contrib


def _mol(out_f, out_b_rev, w_o, b_o):
    w0, w1 = w_o[:H], w_o[H:]
    return pl.pallas_call(
        _mol_body,
        grid=(L,),
        in_specs=[pl.BlockSpec((1, NM, H), lambda t: (t, 0, 0)),
                  pl.BlockSpec((1, NM, H), lambda t: (L - 1 - t, 0, 0)),
                  pl.BlockSpec((H, H), lambda t: (0, 0)),
                  pl.BlockSpec((H, H), lambda t: (0, 0)),
                  pl.BlockSpec((1, H), lambda t: (0, 0))],
        out_specs=pl.BlockSpec((NM, H), lambda t: (0, 0)),
        out_shape=jax.ShapeDtypeStruct((NM, H), F32),
    )(out_f, out_b_rev, w0, w1, b_o.reshape(1, H))


def kernel(f_atoms, f_bonds, a2b, b2a, b2revb, W_i_atom, W_i_bond, W_h, W_lr,
           W_o, b_o, gru_bias, W_ih_fwd, W_hh_fwd, b_ih_fwd, b_hh_fwd,
           W_ih_bwd, W_hh_bwd, b_ih_bwd, b_hh_bwd):
    A = f_atoms.shape[0]
    Bn = f_bonds.shape[0]
    depth_m1 = W_h.shape[0]

    fa = jnp.pad(f_atoms, ((0, AP - A), (0, 0)))
    fbt = jnp.pad(f_bonds.T, ((0, 0), (0, BP - Bn)))
    a2b_flat = jnp.pad(a2b.astype(jnp.int32),
                       ((0, AP - A), (0, NB8 - NB))).reshape(-1)
    b2a_flat = jnp.pad(b2a.astype(jnp.int32), (0, BP - Bn))
    b2r_flat = jnp.pad(b2revb.astype(jnp.int32), (0, BP - Bn))

    ia = _relu_mm(fa, W_i_atom, blk=2048)
    ib = _relu_mm_t(fbt, W_i_bond, blk=4096)

    ma, mb = ia, ib
    for d in range(depth_m1):
        ma = _sc_atom(mb, ma, a2b_flat, add_matom=True)
        pre = _sc_bond(ma, mb, b2a_flat, b2r_flat)
        mb = _depth_mm(pre, W_h[d], ib, blk=4096)
    agg = _sc_atom(mb, ma, a2b_flat, add_matom=False)

    hidden = _readout(agg, ma, ia, W_lr)
    hid_tm = hidden[1:1 + NM * L].reshape(NM, L, H).transpose(1, 0, 2)
    h0 = _h0(hid_tm)
    out_f, out_b_rev = _gru(hid_tm, h0,
                            W_ih_fwd, W_hh_fwd, b_ih_fwd, b_hh_fwd,
                            W_ih_bwd, W_hh_bwd, b_ih_bwd, b_hh_bwd, gru_bias)
    return _mol(out_f, out_b_rev, W_o, b_o)


# speedup vs baseline: 11.2208x; 11.2208x over previous
import functools

import jax
import jax.numpy as jnp
from jax import lax
from jax.experimental import pallas as pl
from jax.experimental.pallas import tpu as pltpu
from jax.experimental.pallas import tpu_sc as plsc

H = 128
NB = 6
NM = 1000
L = 50
NC, NS = 2, 16
NW = NC * NS

AP = 51200
BP = 200704

A_PW, A_SB, A_NSB = 1600, 16, 100
NB8 = NB
B_PW, B_SB, B_NSB = 6272, 128, 49

F32 = jnp.float32


DEPTH_RING = 3


def _sc_atom_body(add_matom, mb_hbm, ma_hbm, a2b_hbm, out_hbm,
                  idx_v, r0_v, r1_v, r2_v, ma0_v, ma1_v, ma2_v,
                  o0_v, o1_v, o2_v, g0, g1, g2, s0, s1, s2):
    wid = lax.axis_index("s") * NC + lax.axis_index("c")
    ibase = wid * A_PW * NB8
    pltpu.sync_copy(a2b_hbm.at[pl.ds(ibase, A_PW * NB8)], idx_v)
    rows = (r0_v, r1_v, r2_v)
    outs = (o0_v, o1_v, o2_v)
    mas = (ma0_v, ma1_v, ma2_v)
    gsem = (g0, g1, g2)
    ssem = (s0, s1, s2)
    nidx = A_SB * NB8

    def fire(i, b):
        pltpu.async_copy(mb_hbm.at[idx_v.at[pl.ds(i * nidx, nidx)]],
                         rows[b], gsem[b])
        if add_matom:
            pltpu.async_copy(ma_hbm.at[pl.ds(wid * A_PW + i * A_SB, A_SB)],
                             mas[b], gsem[b])

    def compute(i, b):
        abase = wid * A_PW + i * A_SB
        @pl.when(i >= DEPTH_RING)
        def _():
            pltpu.make_async_copy(outs[b], out_hbm.at[pl.ds(abase, A_SB)],
                                  ssem[b]).wait()
        pltpu.make_async_copy(mb_hbm.at[pl.ds(0, nidx)], rows[b],
                              gsem[b]).wait()
        if add_matom:
            pltpu.make_async_copy(ma_hbm.at[pl.ds(0, A_SB)], mas[b],
                                  gsem[b]).wait()
        o_v = outs[b]
        ma_v = mas[b]
        r_v = rows[b]

        def per_atom(a, carry):
            for ch in range(H // 16):
                sl = pl.ds(ch * 16, 16)
                v = r_v[a * NB8, sl]
                ssum = v
                smax = v
                for j in range(1, NB):
                    vj = r_v[a * NB8 + j, sl]
                    ssum = ssum + vj
                    smax = jnp.maximum(smax, vj)
                res = ssum * smax
                if add_matom:
                    res = res + ma_v[a, sl]
                o_v[a, sl] = res
            return carry

        lax.fori_loop(0, A_SB, per_atom, 0)
        pltpu.async_copy(o_v, out_hbm.at[pl.ds(abase, A_SB)], ssem[b])

    for p in range(DEPTH_RING - 1):
        fire(p, p)

    def step3(i3, carry):
        for k in range(DEPTH_RING):
            i = i3 * DEPTH_RING + k

            @pl.when(i + DEPTH_RING - 1 < A_NSB)
            def _():
                fire(i + DEPTH_RING - 1, (k + DEPTH_RING - 1) % DEPTH_RING)

            compute(i, k)
        return carry

    lax.fori_loop(0, A_NSB // DEPTH_RING, step3, 0)
    tail = (A_NSB // DEPTH_RING) * DEPTH_RING
    for i in range(tail, A_NSB):
        compute(i, i % DEPTH_RING)
    for i in range(A_NSB - DEPTH_RING, A_NSB):
        b = i % DEPTH_RING
        abase = wid * A_PW + i * A_SB
        pltpu.make_async_copy(outs[b], out_hbm.at[pl.ds(abase, A_SB)],
                              ssem[b]).wait()


def _sc_atom(mb, ma, a2b_flat, add_matom):
    return pl.kernel(
        functools.partial(_sc_atom_body, add_matom),
        out_type=jax.ShapeDtypeStruct((AP, H), F32),
        mesh=plsc.VectorSubcoreMesh(core_axis_name="c", subcore_axis_name="s"),
        scratch_types=[
            pltpu.VMEM((A_PW * NB8,), jnp.int32),
            pltpu.VMEM((A_SB * NB8, H), F32),
            pltpu.VMEM((A_SB * NB8, H), F32),
            pltpu.VMEM((A_SB * NB8, H), F32),
            pltpu.VMEM((A_SB, H), F32),
            pltpu.VMEM((A_SB, H), F32),
            pltpu.VMEM((A_SB, H), F32),
            pltpu.VMEM((A_SB, H), F32),
            pltpu.VMEM((A_SB, H), F32),
            pltpu.VMEM((A_SB, H), F32),
            pltpu.SemaphoreType.DMA,
            pltpu.SemaphoreType.DMA,
            pltpu.SemaphoreType.DMA,
            pltpu.SemaphoreType.DMA,
            pltpu.SemaphoreType.DMA,
            pltpu.SemaphoreType.DMA,
        ],
    )(mb, ma, a2b_flat)


def _sc_bond_body(ma_hbm, mb_hbm, b2a_hbm, b2r_hbm, out_hbm,
                  idxa_v, idxr_v, a0_v, a1_v, r0_v, r1_v, o0_v, o1_v,
                  g0, g1, s0, s1):
    wid = lax.axis_index("s") * NC + lax.axis_index("c")
    ibase = wid * B_PW
    pltpu.sync_copy(b2a_hbm.at[pl.ds(ibase, B_PW)], idxa_v)
    pltpu.sync_copy(b2r_hbm.at[pl.ds(ibase, B_PW)], idxr_v)
    ags = (a0_v, a1_v)
    rvs = (r0_v, r1_v)
    outs = (o0_v, o1_v)
    gsem = (g0, g1)
    ssem = (s0, s1)

    def fire(i, b):
        off = i * B_SB
        pltpu.async_copy(ma_hbm.at[idxa_v.at[pl.ds(off, B_SB)]], ags[b], gsem[b])
        pltpu.async_copy(mb_hbm.at[idxr_v.at[pl.ds(off, B_SB)]], rvs[b], gsem[b])

    def compute(i, b):
        bbase = wid * B_PW + i * B_SB

        @pl.when(i >= 2)
        def _():
            pltpu.make_async_copy(outs[b], out_hbm.at[pl.ds(bbase, B_SB)],
                                  ssem[b]).wait()

        desc = pltpu.make_async_copy(ma_hbm.at[pl.ds(0, B_SB)], ags[b], gsem[b])
        desc.wait()
        desc.wait()
        o_v = outs[b]

        def inner(r, c):
            for ch in range(H // 16):
                sl = pl.ds(ch * 16, 16)
                o_v[r, sl] = ags[b][r, sl] - rvs[b][r, sl]
            return c

        lax.fori_loop(0, B_SB, inner, 0)
        pltpu.async_copy(o_v, out_hbm.at[pl.ds(bbase, B_SB)], ssem[b])

    fire(0, 0)

    def step2(i2, carry):
        for b in range(2):
            i = i2 * 2 + b

            @pl.when(i + 1 < B_NSB)
            def _():
                fire(i + 1, (b + 1) % 2)

            @pl.when(i < B_NSB)
            def _():
                compute(i, b)
        return carry

    lax.fori_loop(0, (B_NSB + 1) // 2, step2, 0)
    for b in range(2):
        bbase = wid * B_PW + (B_NSB - 2 + b) * B_SB
        pltpu.make_async_copy(outs[b], out_hbm.at[pl.ds(bbase, B_SB)],
                              ssem[b]).wait()


def _sc_bond(ma, mb, b2a_flat, b2r_flat):
    return pl.kernel(
        _sc_bond_body,
        out_type=jax.ShapeDtypeStruct((BP, H), F32),
        mesh=plsc.VectorSubcoreMesh(core_axis_name="c", subcore_axis_name="s"),
        scratch_types=[
            pltpu.VMEM((B_PW,), jnp.int32),
            pltpu.VMEM((B_PW,), jnp.int32),
            pltpu.VMEM((B_SB, H), F32),
            pltpu.VMEM((B_SB, H), F32),
            pltpu.VMEM((B_SB, H), F32),
            pltpu.VMEM((B_SB, H), F32),
            pltpu.VMEM((B_SB, H), F32),
            pltpu.VMEM((B_SB, H), F32),
            pltpu.SemaphoreType.DMA,
            pltpu.SemaphoreType.DMA,
            pltpu.SemaphoreType.DMA,
            pltpu.SemaphoreType.DMA,
        ],
    )(ma, mb, b2a_flat, b2r_flat)


def _relu_mm_body(x_ref, w_ref, o_ref):
    o_ref[...] = jax.nn.relu(
        jnp.dot(x_ref[...], w_ref[...], preferred_element_type=F32))


def _relu_mm(x, w, blk):
    n, k = x.shape
    m = w.shape[1]
    return pl.pallas_call(
        _relu_mm_body,
        grid=(n // blk,),
        in_specs=[pl.BlockSpec((blk, k), lambda i: (i, 0)),
                  pl.BlockSpec((k, m), lambda i: (0, 0))],
        out_specs=pl.BlockSpec((blk, m), lambda i: (i, 0)),
        out_shape=jax.ShapeDtypeStruct((n, m), F32),
    )(x, w)


def _relu_mm_t_body(xt_ref, w_ref, o_ref):
    o_ref[...] = jax.nn.relu(lax.dot_general(
        xt_ref[...], w_ref[...], (((0,), (0,)), ((), ())),
        preferred_element_type=F32))


def _relu_mm_t(xt, w, blk):
    k, n = xt.shape
    m = w.shape[1]
    return pl.pallas_call(
        _relu_mm_t_body,
        grid=(n // blk,),
        in_specs=[pl.BlockSpec((k, blk), lambda i: (0, i)),
                  pl.BlockSpec((k, m), lambda i: (0, 0))],
        out_specs=pl.BlockSpec((blk, m), lambda i: (i, 0)),
        out_shape=jax.ShapeDtypeStruct((n, m), F32),
    )(xt, w)


def _depth_mm_body(x_ref, w_ref, b_ref, o_ref):
    o_ref[...] = jax.nn.relu(
        b_ref[...] + jnp.dot(x_ref[...], w_ref[...], preferred_element_type=F32))


def _depth_mm(pre, w, ib, blk):
    n, k = pre.shape
    m = w.shape[1]
    return pl.pallas_call(
        _depth_mm_body,
        grid=(n // blk,),
        in_specs=[pl.BlockSpec((blk, k), lambda i: (i, 0)),
                  pl.BlockSpec((k, m), lambda i: (0, 0)),
                  pl.BlockSpec((blk, m), lambda i: (i, 0))],
        out_specs=pl.BlockSpec((blk, m), lambda i: (i, 0)),
        out_shape=jax.ShapeDtypeStruct((n, m), F32),
    )(pre, w, ib)


def _readout_body(a_ref, m_ref, i_ref, w0, w1, w2, o_ref):
    o_ref[...] = (jnp.dot(a_ref[...], w0[...], preferred_element_type=F32)
                  + jnp.dot(m_ref[...], w1[...], preferred_element_type=F32)
                  + jnp.dot(i_ref[...], w2[...], preferred_element_type=F32))


def _readout(agg, ma, ia, w_lr, blk=2048):
    n = agg.shape[0]
    w0, w1, w2 = w_lr[:H], w_lr[H:2 * H], w_lr[2 * H:]
    return pl.pallas_call(
        _readout_body,
        grid=(n // blk,),
        in_specs=[pl.BlockSpec((blk, H), lambda i: (i, 0)),
                  pl.BlockSpec((blk, H), lambda i: (i, 0)),
                  pl.BlockSpec((blk, H), lambda i: (i, 0)),
                  pl.BlockSpec((H, H), lambda i: (0, 0)),
                  pl.BlockSpec((H, H), lambda i: (0, 0)),
                  pl.BlockSpec((H, H), lambda i: (0, 0))],
        out_specs=pl.BlockSpec((blk, H), lambda i: (i, 0)),
        out_shape=jax.ShapeDtypeStruct((n, H), F32),
    )(agg, ma, ia, w0, w1, w2)


def _h0_body(hid_ref, o_ref):
    t = pl.program_id(0)

    @pl.when(t == 0)
    def _():
        o_ref[...] = hid_ref[0]

    @pl.when(t > 0)
    def _():
        o_ref[...] = jnp.maximum(o_ref[...], hid_ref[0])


def _h0(hid_tm):
    return pl.pallas_call(
        _h0_body,
        grid=(L,),
        in_specs=[pl.BlockSpec((1, NM, H), lambda t: (t, 0, 0))],
        out_specs=pl.BlockSpec((NM, H), lambda t: (0, 0)),
        out_shape=jax.ShapeDtypeStruct((NM, H), F32),
    )(hid_tm)


def _gru_body(hf_ref, hb_ref, h0_ref, wih_f, whh_f, wih_b, whh_b,
              bih_f, bhh_f, bih_b, bhh_b, gb_ref,
              of_ref, ob_ref, hf, hb):
    t = pl.program_id(0)

    @pl.when(t == 0)
    def _():
        hf[...] = h0_ref[...]
        hb[...] = h0_ref[...]

    gb = gb_ref[...]

    def cell(x, h, wih, whh, bih, bhh):
        gi = jnp.dot(x, wih[...], preferred_element_type=F32) + bih[...]
        gh = jnp.dot(h, whh[...], preferred_element_type=F32) + bhh[...]
        r = jax.nn.sigmoid(gi[:, :H] + gh[:, :H])
        z = jax.nn.sigmoid(gi[:, H:2 * H] + gh[:, H:2 * H])
        n = jnp.tanh(gi[:, 2 * H:] + r * gh[:, 2 * H:])
        return (1.0 - z) * n + z * h

    xf = jax.nn.relu(hf_ref[0] + gb)
    hfn = cell(xf, hf[...], wih_f, whh_f, bih_f, bhh_f)
    hf[...] = hfn
    of_ref[0] = hfn

    xb = jax.nn.relu(hb_ref[0] + gb)
    hbn = cell(xb, hb[...], wih_b, whh_b, bih_b, bhh_b)
    hb[...] = hbn
    ob_ref[0] = hbn


def _gru(hid_tm, h0, wih_f, whh_f, bih_f, bhh_f, wih_b, whh_b, bih_b, bhh_b,
         gru_bias):
    wspec = pl.BlockSpec((H, 3 * H), lambda t: (0, 0))
    bspec = pl.BlockSpec((1, 3 * H), lambda t: (0, 0))
    return pl.pallas_call(
        _gru_body,
        grid=(L,),
        in_specs=[pl.BlockSpec((1, NM, H), lambda t: (t, 0, 0)),
                  pl.BlockSpec((1, NM, H), lambda t: (L - 1 - t, 0, 0)),
                  pl.BlockSpec((NM, H), lambda t: (0, 0)),
                  wspec, wspec, wspec, wspec,
                  bspec, bspec, bspec, bspec,
                  pl.BlockSpec((1, H), lambda t: (0, 0))],
        out_specs=[pl.BlockSpec((1, NM, H), lambda t: (t, 0, 0)),
                   pl.BlockSpec((1, NM, H), lambda t: (t, 0, 0))],
        out_shape=[jax.ShapeDtypeStruct((L, NM, H), F32),
                   jax.ShapeDtypeStruct((L, NM, H), F32)],
        scratch_shapes=[pltpu.VMEM((NM, H), F32),
                        pltpu.VMEM((NM, H), F32)],
    )(hid_tm, hid_tm, h0,
      wih_f, whh_f, wih_b, whh_b,
      bih_f.reshape(1, 3 * H), bhh_f.reshape(1, 3 * H),
      bih_b.reshape(1, 3 * H), bhh_b.reshape(1, 3 * H),
      gru_bias.reshape(1, H))


def _mol_body(of_ref, ob_ref, w0, w1, bo_ref, o_ref):
    t = pl.program_id(0)
    contrib = jax.nn.relu(
        jnp.dot(of_ref[0], w0[...], preferred_element_type=F32)
        + jnp.dot(ob_ref[0], w1[...], preferred_element_type=F32)
        + bo_ref[...]) * (1.0 / L)

    @pl.when(t == 0)
    def _():
        o_ref[...] = contrib

    @pl.when(t > 0)
    def _():
        o_ref[...] = o_ref[...] + contrib


def _mol(out_f, out_b_rev, w_o, b_o):
    w0, w1 = w_o[:H], w_o[H:]
    return pl.pallas_call(
        _mol_body,
        grid=(L,),
        in_specs=[pl.BlockSpec((1, NM, H), lambda t: (t, 0, 0)),
                  pl.BlockSpec((1, NM, H), lambda t: (L - 1 - t, 0, 0)),
                  pl.BlockSpec((H, H), lambda t: (0, 0)),
                  pl.BlockSpec((H, H), lambda t: (0, 0)),
                  pl.BlockSpec((1, H), lambda t: (0, 0))],
        out_specs=pl.BlockSpec((NM, H), lambda t: (0, 0)),
        out_shape=jax.ShapeDtypeStruct((NM, H), F32),
    )(out_f, out_b_rev, w0, w1, b_o.reshape(1, H))


def kernel(f_atoms, f_bonds, a2b, b2a, b2revb, W_i_atom, W_i_bond, W_h, W_lr,
           W_o, b_o, gru_bias, W_ih_fwd, W_hh_fwd, b_ih_fwd, b_hh_fwd,
           W_ih_bwd, W_hh_bwd, b_ih_bwd, b_hh_bwd):
    A = f_atoms.shape[0]
    Bn = f_bonds.shape[0]
    depth_m1 = W_h.shape[0]

    fa = jnp.pad(f_atoms, ((0, AP - A), (0, 0)))
    fbt = jnp.pad(f_bonds.T, ((0, 0), (0, BP - Bn)))
    pad_a2b = jnp.arange((AP - A) * NB8, dtype=jnp.int32) % Bn
    a2b_flat = jnp.concatenate(
        [a2b.astype(jnp.int32).reshape(-1), pad_a2b])
    pad_b = jnp.arange(BP - Bn, dtype=jnp.int32)
    b2a_flat = jnp.concatenate([b2a.astype(jnp.int32), pad_b % A])
    b2r_flat = jnp.concatenate([b2revb.astype(jnp.int32), pad_b])

    ia = _relu_mm(fa, W_i_atom, blk=2048)
    ib = _relu_mm_t(fbt, W_i_bond, blk=4096)

    ma, mb = ia, ib
    for d in range(depth_m1):
        ma = _sc_atom(mb, ma, a2b_flat, add_matom=True)
        pre = _sc_bond(ma, mb, b2a_flat, b2r_flat)
        mb = _depth_mm(pre, W_h[d], ib, blk=4096)
    agg = _sc_atom(mb, ma, a2b_flat, add_matom=False)

    hidden = _readout(agg, ma, ia, W_lr)
    hid_tm = hidden[1:1 + NM * L].reshape(NM, L, H).transpose(1, 0, 2)
    h0 = _h0(hid_tm)
    out_f, out_b_rev = _gru(hid_tm, h0,
                            W_ih_fwd, W_hh_fwd, b_ih_fwd, b_hh_fwd,
                            W_ih_bwd, W_hh_bwd, b_ih_bwd, b_hh_bwd, gru_bias)
    return _mol(out_f, out_b_rev, W_o, b_o)


# speedup vs baseline: 11.6007x; 1.0339x over previous
import functools

import jax
import jax.numpy as jnp
from jax import lax
from jax.experimental import pallas as pl
from jax.experimental.pallas import tpu as pltpu
from jax.experimental.pallas import tpu_sc as plsc

H = 128
NB = 6
NM = 1000
L = 50
NC, NS = 2, 16
NW = NC * NS

AP = 51200
BP = 200704

A_PW, A_SB, A_NSB = 1600, 40, 40
NB8 = NB
QIDX = A_SB * NB // 2
B_PW, B_SB, B_NSB = 6272, 128, 49

F32 = jnp.float32


DEPTH_RING = 2


def _sc_atom_body(add_matom, mb_hbm, ma_hbm, a2b_hbm, out_hbm,
                  idx_v, r0_v, r1_v, ma0_v, ma1_v,
                  o0_v, o1_v, g0, g1, s0, s1):
    wid = lax.axis_index("s") * NC + lax.axis_index("c")
    ibase = wid * A_PW * NB8
    pltpu.sync_copy(a2b_hbm.at[pl.ds(ibase, A_PW * NB8)], idx_v)
    rows = (r0_v, r1_v)
    outs = (o0_v, o1_v)
    mas = (ma0_v, ma1_v)
    gsem = (g0, g1)
    ssem = (s0, s1)
    nidx = A_SB * NB8

    def fire(i, b):
        off = i * nidx
        pltpu.async_copy(mb_hbm.at[idx_v.at[pl.ds(off, QIDX)]],
                         rows[b].at[pl.ds(0, QIDX)], gsem[b])
        pltpu.async_copy(mb_hbm.at[idx_v.at[pl.ds(off + QIDX, QIDX)]],
                         rows[b].at[pl.ds(QIDX, QIDX)], gsem[b])
        if add_matom:
            pltpu.async_copy(ma_hbm.at[pl.ds(wid * A_PW + i * A_SB, A_SB)],
                             mas[b], gsem[b])

    def compute(i, b):
        abase = wid * A_PW + i * A_SB
        @pl.when(i >= DEPTH_RING)
        def _():
            pltpu.make_async_copy(outs[b], out_hbm.at[pl.ds(abase, A_SB)],
                                  ssem[b]).wait()
        desc = pltpu.make_async_copy(mb_hbm.at[pl.ds(0, QIDX)],
                                     rows[b].at[pl.ds(0, QIDX)], gsem[b])
        desc.wait()
        desc.wait()
        if add_matom:
            pltpu.make_async_copy(ma_hbm.at[pl.ds(0, A_SB)], mas[b],
                                  gsem[b]).wait()
        o_v = outs[b]
        ma_v = mas[b]
        r_v = rows[b]

        def per_atom(a, carry):
            for ch in range(H // 16):
                sl = pl.ds(ch * 16, 16)
                v = r_v[a * NB8, sl]
                ssum = v
                smax = v
                for j in range(1, NB):
                    vj = r_v[a * NB8 + j, sl]
                    ssum = ssum + vj
                    smax = jnp.maximum(smax, vj)
                res = ssum * smax
                if add_matom:
                    res = res + ma_v[a, sl]
                o_v[a, sl] = res
            return carry

        lax.fori_loop(0, A_SB, per_atom, 0)
        pltpu.async_copy(o_v, out_hbm.at[pl.ds(abase, A_SB)], ssem[b])

    for p in range(DEPTH_RING - 1):
        fire(p, p)

    def step3(i3, carry):
        for k in range(DEPTH_RING):
            i = i3 * DEPTH_RING + k

            @pl.when(i + DEPTH_RING - 1 < A_NSB)
            def _():
                fire(i + DEPTH_RING - 1, (k + DEPTH_RING - 1) % DEPTH_RING)

            compute(i, k)
        return carry

    lax.fori_loop(0, A_NSB // DEPTH_RING, step3, 0)
    for i in range(A_NSB - DEPTH_RING, A_NSB):
        b = i % DEPTH_RING
        abase = wid * A_PW + i * A_SB
        pltpu.make_async_copy(outs[b], out_hbm.at[pl.ds(abase, A_SB)],
                              ssem[b]).wait()


def _sc_atom(mb, ma, a2b_flat, add_matom):
    return pl.kernel(
        functools.partial(_sc_atom_body, add_matom),
        out_type=jax.ShapeDtypeStruct((AP, H), F32),
        mesh=plsc.VectorSubcoreMesh(core_axis_name="c", subcore_axis_name="s"),
        scratch_types=[
            pltpu.VMEM((A_PW * NB8,), jnp.int32),
            pltpu.VMEM((A_SB * NB8, H), F32),
            pltpu.VMEM((A_SB * NB8, H), F32),
            pltpu.VMEM((A_SB, H), F32),
            pltpu.VMEM((A_SB, H), F32),
            pltpu.VMEM((A_SB, H), F32),
            pltpu.VMEM((A_SB, H), F32),
            pltpu.SemaphoreType.DMA,
            pltpu.SemaphoreType.DMA,
            pltpu.SemaphoreType.DMA,
            pltpu.SemaphoreType.DMA,
        ],
    )(mb, ma, a2b_flat)


def _sc_bond_body(ma_hbm, mb_hbm, b2a_hbm, b2r_hbm, out_hbm,
                  idxa_v, idxr_v, a0_v, a1_v, r0_v, r1_v, o0_v, o1_v,
                  g0, g1, s0, s1):
    wid = lax.axis_index("s") * NC + lax.axis_index("c")
    ibase = wid * B_PW
    pltpu.sync_copy(b2a_hbm.at[pl.ds(ibase, B_PW)], idxa_v)
    pltpu.sync_copy(b2r_hbm.at[pl.ds(ibase, B_PW)], idxr_v)
    ags = (a0_v, a1_v)
    rvs = (r0_v, r1_v)
    outs = (o0_v, o1_v)
    gsem = (g0, g1)
    ssem = (s0, s1)

    def fire(i, b):
        off = i * B_SB
        pltpu.async_copy(ma_hbm.at[idxa_v.at[pl.ds(off, B_SB)]], ags[b], gsem[b])
        pltpu.async_copy(mb_hbm.at[idxr_v.at[pl.ds(off, B_SB)]], rvs[b], gsem[b])

    def compute(i, b):
        bbase = wid * B_PW + i * B_SB

        @pl.when(i >= 2)
        def _():
            pltpu.make_async_copy(outs[b], out_hbm.at[pl.ds(bbase, B_SB)],
                                  ssem[b]).wait()

        desc = pltpu.make_async_copy(ma_hbm.at[pl.ds(0, B_SB)], ags[b], gsem[b])
        desc.wait()
        desc.wait()
        o_v = outs[b]

        def inner(r, c):
            for ch in range(H // 16):
                sl = pl.ds(ch * 16, 16)
                o_v[r, sl] = ags[b][r, sl] - rvs[b][r, sl]
            return c

        lax.fori_loop(0, B_SB, inner, 0)
        pltpu.async_copy(o_v, out_hbm.at[pl.ds(bbase, B_SB)], ssem[b])

    fire(0, 0)

    def step2(i2, carry):
        for b in range(2):
            i = i2 * 2 + b

            @pl.when(i + 1 < B_NSB)
            def _():
                fire(i + 1, (b + 1) % 2)

            @pl.when(i < B_NSB)
            def _():
                compute(i, b)
        return carry

    lax.fori_loop(0, (B_NSB + 1) // 2, step2, 0)
    for b in range(2):
        bbase = wid * B_PW + (B_NSB - 2 + b) * B_SB
        pltpu.make_async_copy(outs[b], out_hbm.at[pl.ds(bbase, B_SB)],
                              ssem[b]).wait()


def _sc_bond(ma, mb, b2a_flat, b2r_flat):
    return pl.kernel(
        _sc_bond_body,
        out_type=jax.ShapeDtypeStruct((BP, H), F32),
        mesh=plsc.VectorSubcoreMesh(core_axis_name="c", subcore_axis_name="s"),
        scratch_types=[
            pltpu.VMEM((B_PW,), jnp.int32),
            pltpu.VMEM((B_PW,), jnp.int32),
            pltpu.VMEM((B_SB, H), F32),
            pltpu.VMEM((B_SB, H), F32),
            pltpu.VMEM((B_SB, H), F32),
            pltpu.VMEM((B_SB, H), F32),
            pltpu.VMEM((B_SB, H), F32),
            pltpu.VMEM((B_SB, H), F32),
            pltpu.SemaphoreType.DMA,
            pltpu.SemaphoreType.DMA,
            pltpu.SemaphoreType.DMA,
            pltpu.SemaphoreType.DMA,
        ],
    )(ma, mb, b2a_flat, b2r_flat)


def _relu_mm_body(x_ref, w_ref, o_ref):
    o_ref[...] = jax.nn.relu(
        jnp.dot(x_ref[...], w_ref[...], preferred_element_type=F32))


def _relu_mm(x, w, blk, n_out=None):
    n, k = x.shape
    m = w.shape[1]
    n_out = n_out or n
    return pl.pallas_call(
        _relu_mm_body,
        grid=(n_out // blk,),
        in_specs=[pl.BlockSpec((blk, k), lambda i: (i, 0)),
                  pl.BlockSpec((k, m), lambda i: (0, 0))],
        out_specs=pl.BlockSpec((blk, m), lambda i: (i, 0)),
        out_shape=jax.ShapeDtypeStruct((n_out, m), F32),
    )(x, w)


def _relu_mm_t_body(xt_ref, w_ref, o_ref):
    o_ref[...] = jax.nn.relu(lax.dot_general(
        xt_ref[...], w_ref[...], (((0,), (0,)), ((), ())),
        preferred_element_type=F32))


def _relu_mm_t(xt, w, blk, n_out=None):
    k, n = xt.shape
    m = w.shape[1]
    n_out = n_out or n
    return pl.pallas_call(
        _relu_mm_t_body,
        grid=(n_out // blk,),
        in_specs=[pl.BlockSpec((k, blk), lambda i: (0, i)),
                  pl.BlockSpec((k, m), lambda i: (0, 0))],
        out_specs=pl.BlockSpec((blk, m), lambda i: (i, 0)),
        out_shape=jax.ShapeDtypeStruct((n_out, m), F32),
    )(xt, w)


def _depth_mm_body(x_ref, w_ref, b_ref, o_ref):
    o_ref[...] = jax.nn.relu(
        b_ref[...] + jnp.dot(x_ref[...], w_ref[...], preferred_element_type=F32))


def _depth_mm(pre, w, ib, blk):
    n, k = pre.shape
    m = w.shape[1]
    return pl.pallas_call(
        _depth_mm_body,
        grid=(n // blk,),
        in_specs=[pl.BlockSpec((blk, k), lambda i: (i, 0)),
                  pl.BlockSpec((k, m), lambda i: (0, 0)),
                  pl.BlockSpec((blk, m), lambda i: (i, 0))],
        out_specs=pl.BlockSpec((blk, m), lambda i: (i, 0)),
        out_shape=jax.ShapeDtypeStruct((n, m), F32),
    )(pre, w, ib)


def _readout_body(a_ref, m_ref, i_ref, w0, w1, w2, o_ref):
    o_ref[...] = (jnp.dot(a_ref[...], w0[...], preferred_element_type=F32)
                  + jnp.dot(m_ref[...], w1[...], preferred_element_type=F32)
                  + jnp.dot(i_ref[...], w2[...], preferred_element_type=F32))


def _readout(agg, ma, ia, w_lr, blk=2048):
    n = agg.shape[0]
    w0, w1, w2 = w_lr[:H], w_lr[H:2 * H], w_lr[2 * H:]
    return pl.pallas_call(
        _readout_body,
        grid=(n // blk,),
        in_specs=[pl.BlockSpec((blk, H), lambda i: (i, 0)),
                  pl.BlockSpec((blk, H), lambda i: (i, 0)),
                  pl.BlockSpec((blk, H), lambda i: (i, 0)),
                  pl.BlockSpec((H, H), lambda i: (0, 0)),
                  pl.BlockSpec((H, H), lambda i: (0, 0)),
                  pl.BlockSpec((H, H), lambda i: (0, 0))],
        out_specs=pl.BlockSpec((blk, H), lambda i: (i, 0)),
        out_shape=jax.ShapeDtypeStruct((n, H), F32),
    )(agg, ma, ia, w0, w1, w2)


def _h0_body(hid_ref, o_ref):
    t = pl.program_id(0)

    @pl.when(t == 0)
    def _():
        o_ref[...] = hid_ref[0]

    @pl.when(t > 0)
    def _():
        o_ref[...] = jnp.maximum(o_ref[...], hid_ref[0])


def _h0(hid_tm):
    return pl.pallas_call(
        _h0_body,
        grid=(L,),
        in_specs=[pl.BlockSpec((1, NM, H), lambda t: (t, 0, 0))],
        out_specs=pl.BlockSpec((NM, H), lambda t: (0, 0)),
        out_shape=jax.ShapeDtypeStruct((NM, H), F32),
    )(hid_tm)


def _gru_body(hf_ref, hb_ref, h0_ref, wih_f, whh_f, wih_b, whh_b,
              bih_f, bhh_f, bih_b, bhh_b, gb_ref,
              of_ref, ob_ref, hf, hb):
    t = pl.program_id(0)

    @pl.when(t == 0)
    def _():
        hf[...] = h0_ref[...]
        hb[...] = h0_ref[...]

    gb = gb_ref[...]

    def cell(x, h, wih, whh, bih, bhh):
        gi = jnp.dot(x, wih[...], preferred_element_type=F32) + bih[...]
        gh = jnp.dot(h, whh[...], preferred_element_type=F32) + bhh[...]
        r = jax.nn.sigmoid(gi[:, :H] + gh[:, :H])
        z = jax.nn.sigmoid(gi[:, H:2 * H] + gh[:, H:2 * H])
        n = jnp.tanh(gi[:, 2 * H:] + r * gh[:, 2 * H:])
        return (1.0 - z) * n + z * h

    xf = jax.nn.relu(hf_ref[0] + gb)
    hfn = cell(xf, hf[...], wih_f, whh_f, bih_f, bhh_f)
    hf[...] = hfn
    of_ref[0] = hfn

    xb = jax.nn.relu(hb_ref[0] + gb)
    hbn = cell(xb, hb[...], wih_b, whh_b, bih_b, bhh_b)
    hb[...] = hbn
    ob_ref[0] = hbn


def _gru(hid_tm, h0, wih_f, whh_f, bih_f, bhh_f, wih_b, whh_b, bih_b, bhh_b,
         gru_bias):
    wspec = pl.BlockSpec((H, 3 * H), lambda t: (0, 0))
    bspec = pl.BlockSpec((1, 3 * H), lambda t: (0, 0))
    return pl.pallas_call(
        _gru_body,
        grid=(L,),
        in_specs=[pl.BlockSpec((1, NM, H), lambda t: (t, 0, 0)),
                  pl.BlockSpec((1, NM, H), lambda t: (L - 1 - t, 0, 0)),
                  pl.BlockSpec((NM, H), lambda t: (0, 0)),
                  wspec, wspec, wspec, wspec,
                  bspec, bspec, bspec, bspec,
                  pl.BlockSpec((1, H), lambda t: (0, 0))],
        out_specs=[pl.BlockSpec((1, NM, H), lambda t: (t, 0, 0)),
                   pl.BlockSpec((1, NM, H), lambda t: (t, 0, 0))],
        out_shape=[jax.ShapeDtypeStruct((L, NM, H), F32),
                   jax.ShapeDtypeStruct((L, NM, H), F32)],
        scratch_shapes=[pltpu.VMEM((NM, H), F32),
                        pltpu.VMEM((NM, H), F32)],
    )(hid_tm, hid_tm, h0,
      wih_f, whh_f, wih_b, whh_b,
      bih_f.reshape(1, 3 * H), bhh_f.reshape(1, 3 * H),
      bih_b.reshape(1, 3 * H), bhh_b.reshape(1, 3 * H),
      gru_bias.reshape(1, H))


def _mol_body(of_ref, ob_ref, w0, w1, bo_ref, o_ref):
    t = pl.program_id(0)
    contrib = jax.nn.relu(
        jnp.dot(of_ref[0], w0[...], preferred_element_type=F32)
        + jnp.dot(ob_ref[0], w1[...], preferred_element_type=F32)
        + bo_ref[...]) * (1.0 / L)

    @pl.when(t == 0)
    def _():
        o_ref[...] = contrib

    @pl.when(t > 0)
    def _():
        o_ref[...] = o_ref[...] + contrib


def _mol(out_f, out_b_rev, w_o, b_o):
    w0, w1 = w_o[:H], w_o[H:]
    return pl.pallas_call(
        _mol_body,
        grid=(L,),
        in_specs=[pl.BlockSpec((1, NM, H), lambda t: (t, 0, 0)),
                  pl.BlockSpec((1, NM, H), lambda t: (L - 1 - t, 0, 0)),
                  pl.BlockSpec((H, H), lambda t: (0, 0)),
                  pl.BlockSpec((H, H), lambda t: (0, 0)),
                  pl.BlockSpec((1, H), lambda t: (0, 0))],
        out_specs=pl.BlockSpec((NM, H), lambda t: (0, 0)),
        out_shape=jax.ShapeDtypeStruct((NM, H), F32),
    )(out_f, out_b_rev, w0, w1, b_o.reshape(1, H))


def kernel(f_atoms, f_bonds, a2b, b2a, b2revb, W_i_atom, W_i_bond, W_h, W_lr,
           W_o, b_o, gru_bias, W_ih_fwd, W_hh_fwd, b_ih_fwd, b_hh_fwd,
           W_ih_bwd, W_hh_bwd, b_ih_bwd, b_hh_bwd):
    A = f_atoms.shape[0]
    Bn = f_bonds.shape[0]
    depth_m1 = W_h.shape[0]

    fbt = f_bonds.T
    pad_a2b = jnp.arange((AP - A) * NB8, dtype=jnp.int32) % Bn
    a2b_flat = jnp.concatenate(
        [a2b.astype(jnp.int32).reshape(-1), pad_a2b])
    pad_b = jnp.arange(BP - Bn, dtype=jnp.int32)
    b2a_flat = jnp.concatenate([b2a.astype(jnp.int32), pad_b % A])
    b2r_flat = jnp.concatenate([b2revb.astype(jnp.int32), pad_b])

    ia = _relu_mm(f_atoms, W_i_atom, blk=2048, n_out=AP)
    ib = _relu_mm_t(fbt, W_i_bond, blk=4096, n_out=BP)

    ma, mb = ia, ib
    for d in range(depth_m1):
        ma = _sc_atom(mb, ma, a2b_flat, add_matom=True)
        pre = _sc_bond(ma, mb, b2a_flat, b2r_flat)
        mb = _depth_mm(pre, W_h[d], ib, blk=4096)
    agg = _sc_atom(mb, ma, a2b_flat, add_matom=False)

    hidden = _readout(agg, ma, ia, W_lr)
    hid_tm = hidden[1:1 + NM * L].reshape(NM, L, H).transpose(1, 0, 2)
    h0 = _h0(hid_tm)
    out_f, out_b_rev = _gru(hid_tm, h0,
                            W_ih_fwd, W_hh_fwd, b_ih_fwd, b_hh_fwd,
                            W_ih_bwd, W_hh_bwd, b_ih_bwd, b_hh_bwd, gru_bias)
    return _mol(out_f, out_b_rev, W_o, b_o)
